# baseline (speedup 1.0000x reference)
import functools

import jax
import jax.numpy as jnp
from jax import lax
from jax.experimental import pallas as pl
from jax.experimental.pallas import tpu as pltpu
from jax.experimental.pallas import tpu_sc as plsc

N = 100000
N_S = 1000000

_INFO = plsc.get_sparse_core_info()
NC, NSUB, L = _INFO.num_cores, _INFO.num_subcores, _INFO.num_lanes
NW = NC * NSUB

N_SP = 1000448
CHUNK = N_SP // NW
CHUNK_VREGS = CHUNK // L

CAP = 131584
CSLICE = CAP // NW
CSLICE_VREGS = CSLICE // L
CAPB = CAP + 16
STAGE = 31488

OUTPAD = 100352
B_PER_W = OUTPAD // NW

HBITS = 272

_mesh = plsc.VectorSubcoreMesh(core_axis_name="c", subcore_axis_name="s")
_params = pltpu.CompilerParams(use_tc_tiling_on_sc=False,
                              needs_layout_passes=False)


def _wid():
    return lax.axis_index("s") * NC + lax.axis_index("c")


def _mono_u32_host(kv):
    b = lax.bitcast_convert_type(kv, jnp.int32)
    m = b ^ ((b >> 31) | jnp.int32(-2147483648))
    return lax.bitcast_convert_type(m, jnp.uint32)


def _iota16():
    return lax.iota(jnp.int32, 16)


def _make_thresh_hist(shift):
    first = shift == 24

    @functools.partial(
        pl.kernel,
        out_type=jax.ShapeDtypeStruct((NW, HBITS), jnp.int32),
        mesh=_mesh,
        compiler_params=_params,
        scratch_types=[
            pltpu.VMEM((CHUNK,), jnp.uint32),
            pltpu.VMEM((16,), jnp.uint32),
            pltpu.VMEM((HBITS,), jnp.int32),
        ],
    )
    def thresh_hist(keys_hbm, pref_hbm, hists_hbm, keys_v, pref_v, hist_v):
        w = _wid()
        pltpu.sync_copy(keys_hbm.at[pl.ds(w * CHUNK, CHUNK)], keys_v)
        if not first:
            pltpu.sync_copy(pref_hbm, pref_v)
        zeros = jnp.zeros((16,), jnp.int32)
        for j in range(HBITS // 16):
            hist_v[pl.ds(j * 16, 16)] = zeros
        if first:
            pref = jnp.uint32(0)
        else:
            pref = pref_v[...][0]

        def body(i, _):
            u = keys_v[pl.ds(i * 16, 16)]
            digit = ((u >> shift) & jnp.uint32(255)).astype(jnp.int32)
            if not first:
                match = (u >> (shift + 8)) == pref
                digit = jnp.where(match, digit, jnp.int32(256))
            counts, last = plsc.scan_count(digit)
            plsc.addupdate_scatter(hist_v, [digit], counts, mask=last)
            return 0

        lax.fori_loop(0, CHUNK_VREGS, body, 0)
        pltpu.sync_copy(hist_v, hists_hbm.at[w])

    return thresh_hist


_thresh_hists = {s: _make_thresh_hist(s) for s in (24, 16, 8, 0)}


def _pick_byte(hists, n_rem):
    total = jnp.sum(hists[:, :256], axis=0)
    t = jnp.cumsum(total[::-1])[::-1]
    b = jnp.sum((t >= n_rem).astype(jnp.int32)) - 1
    t_above = jnp.concatenate([t[1:], jnp.zeros((1,), t.dtype)])
    n_rem_next = n_rem - t_above[b]
    return b.astype(jnp.uint32), n_rem_next


@functools.partial(
    pl.kernel,
    out_type=jax.ShapeDtypeStruct((NW, 16), jnp.int32),
    mesh=_mesh,
    compiler_params=_params,
    scratch_types=[
        pltpu.VMEM((CHUNK,), jnp.uint32),
        pltpu.VMEM((16,), jnp.uint32),
        pltpu.VMEM((16,), jnp.int32),
    ],
)
def _count_kernel(keys_hbm, tval_hbm, counts_hbm, keys_v, tval_v, out_v):
    w = _wid()
    pltpu.sync_copy(keys_hbm.at[pl.ds(w * CHUNK, CHUNK)], keys_v)
    pltpu.sync_copy(tval_hbm, tval_v)
    tv = tval_v[...][0]

    def body(i, cnt):
        u = keys_v[pl.ds(i * 16, 16)]
        sel = (u >= tv).astype(jnp.int32)
        return cnt + jnp.sum(sel)

    cnt = lax.fori_loop(0, CHUNK_VREGS, body, jnp.int32(0))
    out_v[...] = jnp.broadcast_to(cnt, (16,))
    pltpu.sync_copy(out_v, counts_hbm.at[w])


@functools.partial(
    pl.kernel,
    out_type=(
        jax.ShapeDtypeStruct((CAPB,), jnp.uint32),
        jax.ShapeDtypeStruct((CAPB,), jnp.int32),
    ),
    mesh=_mesh,
    compiler_params=_params,
    scratch_types=[
        pltpu.VMEM((CHUNK,), jnp.uint32),
        pltpu.VMEM((16,), jnp.uint32),
        pltpu.VMEM((16,), jnp.int32),
        pltpu.VMEM((16,), jnp.int32),
        pltpu.VMEM((STAGE,), jnp.uint32),
        pltpu.VMEM((STAGE,), jnp.int32),
        pltpu.VMEM((128,), jnp.int32),
        pltpu.VMEM((128,), jnp.uint32),
        pltpu.VMEM((128,), jnp.int32),
        pltpu.SemaphoreType.DMA,
    ],
)
def _compact_kernel(keys_hbm, tval_hbm, offs_hbm, cval_hbm, cu_hbm, cidx_hbm,
                    keys_v, tval_v, offs_v, cval_v, su_v, si_v,
                    widx_v, wu_v, wi_v, sem):
    w = _wid()
    pltpu.sync_copy(keys_hbm.at[pl.ds(w * CHUNK, CHUNK)], keys_v)
    pltpu.sync_copy(tval_hbm, tval_v)
    pltpu.sync_copy(offs_hbm.at[w], offs_v)
    pltpu.sync_copy(cval_hbm, cval_v)
    tv = tval_v[...][0]
    off = offs_v[...][0]
    cval = cval_v[...][0]
    base = w * CHUNK
    it16 = _iota16()

    def body(i, cnt):
        u = keys_v[pl.ds(i * 16, 16)]
        gidx = base + i * 16 + it16
        mask = u >= tv
        plsc.store_compressed(su_v.at[pl.ds(cnt, 16)], u, mask=mask)
        plsc.store_compressed(si_v.at[pl.ds(cnt, 16)], gidx, mask=mask)
        return cnt + jnp.sum(mask.astype(jnp.int32))

    cnt = lax.fori_loop(0, CHUNK_VREGS, body, jnp.int32(0))

    nwin = (cnt + 127) // 128

    def win_body(j, _):
        jb = j * 128
        for k in range(8):
            pos = jb + k * 16 + it16
            valid = pos < cnt
            dst = jnp.where(valid, off + pos, CAP + it16)
            widx_v[pl.ds(k * 16, 16)] = dst
        c1 = pltpu.async_copy(su_v.at[pl.ds(jb, 128)], cu_hbm.at[widx_v], sem)
        c2 = pltpu.async_copy(si_v.at[pl.ds(jb, 128)], cidx_hbm.at[widx_v], sem)
        c1.wait()
        c2.wait()
        return 0

    lax.fori_loop(0, nwin, win_body, 0)

    zu = jnp.zeros((16,), jnp.uint32)
    zi = jnp.zeros((16,), jnp.int32)
    for k in range(8):
        wu_v[pl.ds(k * 16, 16)] = zu
        wi_v[pl.ds(k * 16, 16)] = zi
    for j in range(8):
        for k in range(8):
            pos = cval + w * 1024 + j * 128 + k * 16 + it16
            dst = jnp.minimum(pos, CAP + it16)
            widx_v[pl.ds(k * 16, 16)] = dst
        c1 = pltpu.async_copy(wu_v, cu_hbm.at[widx_v], sem)
        c2 = pltpu.async_copy(wi_v, cidx_hbm.at[widx_v], sem)
        c1.wait()
        c2.wait()


def _make_sort_hist(shift):
    @functools.partial(
        pl.kernel,
        out_type=jax.ShapeDtypeStruct((NW, HBITS), jnp.int32),
        mesh=_mesh,
        compiler_params=_params,
        scratch_types=[
            pltpu.VMEM((CSLICE,), jnp.uint32),
            pltpu.VMEM((HBITS,), jnp.int32),
        ],
    )
    def sort_hist(cu_hbm, hists_hbm, cu_v, hist_v):
        w = _wid()
        pltpu.sync_copy(cu_hbm.at[pl.ds(w * CSLICE, CSLICE)], cu_v)
        zeros = jnp.zeros((16,), jnp.int32)
        for j in range(HBITS // 16):
            hist_v[pl.ds(j * 16, 16)] = zeros

        def body(i, _):
            u = cu_v[pl.ds(i * 16, 16)]
            digit = ((u >> shift) & jnp.uint32(255)).astype(jnp.int32)
            counts, last = plsc.scan_count(digit)
            plsc.addupdate_scatter(hist_v, [digit], counts, mask=last)
            return 0

        lax.fori_loop(0, CSLICE_VREGS, body, 0)
        pltpu.sync_copy(hist_v, hists_hbm.at[w])

    return sort_hist


def _make_sort_perm(shift):
    nwin = (CSLICE + 127) // 128

    @functools.partial(
        pl.kernel,
        out_type=(
            jax.ShapeDtypeStruct((CAPB,), jnp.uint32),
            jax.ShapeDtypeStruct((CAPB,), jnp.int32),
        ),
        mesh=_mesh,
        compiler_params=_params,
        scratch_types=[
            pltpu.VMEM((nwin * 128,), jnp.uint32),
            pltpu.VMEM((nwin * 128,), jnp.int32),
            pltpu.VMEM((256,), jnp.int32),
            pltpu.VMEM((nwin, 128), jnp.int32),
            pltpu.SemaphoreType.DMA,
        ],
    )
    def sort_perm(cu_hbm, cidx_hbm, bases_hbm, cu2_hbm, cidx2_hbm,
                  cu_v, cidx_v, ctr_v, pos_v, sem):
        w = _wid()
        pltpu.sync_copy(cu_hbm.at[pl.ds(w * CSLICE, CSLICE)],
                        cu_v.at[pl.ds(0, CSLICE)])
        pltpu.sync_copy(cidx_hbm.at[pl.ds(w * CSLICE, CSLICE)],
                        cidx_v.at[pl.ds(0, CSLICE)])
        pltpu.sync_copy(bases_hbm.at[w], ctr_v)
        it16 = _iota16()

        for k in range(1, 8):
            pos_v[nwin - 1, pl.ds(k * 16, 16)] = CAP + it16

        def body(i, _):
            u = cu_v[pl.ds(i * 16, 16)]
            digit = ((u >> shift) & jnp.uint32(255)).astype(jnp.int32)
            counts, last = plsc.scan_count(digit)
            ctr = plsc.load_gather(ctr_v, [digit])
            pos = ctr + counts - 1
            pos_v[i // 8, pl.ds((i % 8) * 16, 16)] = pos
            plsc.addupdate_scatter(ctr_v, [digit], counts, mask=last)
            return 0

        lax.fori_loop(0, CSLICE_VREGS, body, 0)

        for g in range(0, nwin, 8):
            copies = []
            for j in range(g, min(g + 8, nwin)):
                copies.append(pltpu.async_copy(
                    cu_v.at[pl.ds(j * 128, 128)], cu2_hbm.at[pos_v.at[j]], sem))
                copies.append(pltpu.async_copy(
                    cidx_v.at[pl.ds(j * 128, 128)], cidx2_hbm.at[pos_v.at[j]],
                    sem))
            for c in copies:
                c.wait()

    return sort_perm


_sort_hists = {s: _make_sort_hist(s) for s in (0, 8, 16, 24)}
_sort_perms = {s: _make_sort_perm(s) for s in (0, 8, 16, 24)}


def _sort_bases(hists):
    h = hists[:, :256]
    seq = h.T[::-1].reshape(-1)
    ex = jnp.concatenate([jnp.zeros((1,), h.dtype), jnp.cumsum(seq)[:-1]])
    return ex.reshape(256, NW)[::-1].T


@functools.partial(
    pl.kernel,
    out_type=(
        jax.ShapeDtypeStruct((OUTPAD,), jnp.float32),
        jax.ShapeDtypeStruct((OUTPAD,), jnp.float32),
    ),
    mesh=_mesh,
    compiler_params=_params,
    scratch_types=[
        pltpu.VMEM((B_PER_W,), jnp.int32),
        pltpu.VMEM((B_PER_W,), jnp.float32),
        pltpu.VMEM((B_PER_W,), jnp.float32),
        pltpu.SemaphoreType.DMA,
        pltpu.SemaphoreType.DMA,
    ],
)
def _gather_kernel(idx_hbm, xs_hbm, ts_hbm, ox_hbm, ot_hbm,
                   idx_v, xrows_v, trows_v, semx, semt):
    w = _wid()
    base = w * B_PER_W
    pltpu.sync_copy(idx_hbm.at[pl.ds(base, B_PER_W)], idx_v)
    cx = pltpu.async_copy(xs_hbm.at[idx_v], xrows_v, semx)
    ct = pltpu.async_copy(ts_hbm.at[idx_v], trows_v, semt)
    cx.wait()
    ct.wait()
    pltpu.sync_copy(xrows_v, ox_hbm.at[pl.ds(base, B_PER_W)])
    pltpu.sync_copy(trows_v, ot_hbm.at[pl.ds(base, B_PER_W)])


def _splat16(x, dtype):
    return jnp.broadcast_to(x.astype(dtype), (16,))


def kernel(loss, x_s, t_s):
    w = loss.reshape(-1)
    gkey = jax.random.key(42)
    g = jax.random.gumbel(gkey, w.shape, dtype=w.dtype)
    keys = jnp.log(jnp.maximum(w, 1e-30)) + g
    u_mono = _mono_u32_host(keys)
    keys_p = jnp.concatenate(
        [u_mono, jnp.zeros((N_SP - N_S,), jnp.uint32)])

    pref = jnp.uint32(0)
    n_rem = jnp.int32(N)
    for shift in (24, 16, 8, 0):
        hists = _thresh_hists[shift](keys_p, _splat16(pref, jnp.uint32))
        b, n_rem = _pick_byte(hists, n_rem)
        pref = (pref << 8) | b
    tval = pref

    counts = _count_kernel(keys_p, _splat16(tval, jnp.uint32))[:, 0]
    offs = jnp.concatenate(
        [jnp.zeros((1,), jnp.int32), jnp.cumsum(counts)[:-1].astype(jnp.int32)])
    cval = jnp.sum(counts).astype(jnp.int32)
    offs16 = jnp.broadcast_to(offs[:, None], (NW, 16))
    cu, cidx = _compact_kernel(keys_p, _splat16(tval, jnp.uint32), offs16,
                               _splat16(cval, jnp.int32))

    for shift in (0, 8, 16, 24):
        hists = _sort_hists[shift](cu)
        bases = _sort_bases(hists)
        cu, cidx = _sort_perms[shift](cu, cidx, bases)

    xg, tg = _gather_kernel(lax.slice(cidx, (0,), (OUTPAD,)),
                            x_s.reshape(-1), t_s.reshape(-1))
    return (xg[:N].reshape(N, 1), tg[:N].reshape(N, 1))


# speedup vs baseline: 14.7442x; 14.7442x over previous
import functools

import jax
import jax.numpy as jnp
from jax import lax
from jax.experimental import pallas as pl
from jax.experimental.pallas import tpu as pltpu
from jax.experimental.pallas import tpu_sc as plsc

N = 100000
N_S = 1000000

_INFO = plsc.get_sparse_core_info()
NC, NSUB, L = _INFO.num_cores, _INFO.num_subcores, _INFO.num_lanes
NW = NC * NSUB

N_SP = 1000448
CHUNK = N_SP // NW
CHUNK_VREGS = CHUNK // L
CHUNK16 = N_SP // NSUB
CHUNK16_VREGS = CHUNK16 // L

PERCAP = 8224
PERCAP_VREGS = PERCAP // L
CAP16 = NSUB * PERCAP
NWIN = (PERCAP + 127) // 128

OUTPAD = 100352
B_PER_W = OUTPAD // NW
O_PER_W16 = OUTPAD // NSUB

HBITS = 272

_mesh = plsc.VectorSubcoreMesh(core_axis_name="c", subcore_axis_name="s")
_params = pltpu.CompilerParams(use_tc_tiling_on_sc=False,
                               needs_layout_passes=False)


def _mono_u32_host(kv):
    b = lax.bitcast_convert_type(kv, jnp.int32)
    m = b ^ ((b >> 31) | jnp.int32(-2147483648))
    return lax.bitcast_convert_type(m, jnp.uint32)


def _iota16():
    return lax.iota(jnp.int32, 16)


def _make_thresh_hist(shift):
    first = shift == 24

    @functools.partial(
        pl.kernel,
        out_type=jax.ShapeDtypeStruct((NW, HBITS), jnp.int32),
        mesh=_mesh,
        compiler_params=_params,
        scratch_types=[
            pltpu.VMEM((CHUNK,), jnp.uint32),
            pltpu.VMEM((16,), jnp.uint32),
            pltpu.VMEM((HBITS,), jnp.int32),
        ],
    )
    def thresh_hist(keys_hbm, pref_hbm, hists_hbm, keys_v, pref_v, hist_v):
        w = lax.axis_index("s") * NC + lax.axis_index("c")
        pltpu.sync_copy(keys_hbm.at[pl.ds(w * CHUNK, CHUNK)], keys_v)
        if not first:
            pltpu.sync_copy(pref_hbm, pref_v)
        zeros = jnp.zeros((16,), jnp.int32)
        for j in range(HBITS // 16):
            hist_v[pl.ds(j * 16, 16)] = zeros
        if first:
            pref = jnp.uint32(0)
        else:
            pref = pref_v[...][0]

        def body(i, _):
            u = keys_v[pl.ds(i * 16, 16)]
            digit = ((u >> shift) & jnp.uint32(255)).astype(jnp.int32)
            if not first:
                match = (u >> (shift + 8)) == pref
                digit = jnp.where(match, digit, jnp.int32(256))
            counts, last = plsc.scan_count(digit)
            plsc.addupdate_scatter(hist_v, [digit], counts, mask=last)
            return 0

        lax.fori_loop(0, CHUNK_VREGS, body, 0)
        pltpu.sync_copy(hist_v, hists_hbm.at[w])

    return thresh_hist


_thresh_hists = {s: _make_thresh_hist(s) for s in (24, 16, 8)}


def _pick_byte(hists, n_rem):
    total = jnp.sum(hists[:, :256], axis=0)
    t = jnp.cumsum(total[::-1])[::-1]
    b = jnp.sum((t >= n_rem).astype(jnp.int32)) - 1
    t_above = jnp.concatenate([t[1:], jnp.zeros((1,), t.dtype)])
    n_rem_next = n_rem - t_above[b]
    return b.astype(jnp.uint32), n_rem_next


@functools.partial(
    pl.kernel,
    out_type=jax.ShapeDtypeStruct((OUTPAD,), jnp.int32),
    mesh=_mesh,
    compiler_params=_params,
    scratch_types=[
        pltpu.VMEM((CHUNK16,), jnp.uint32),
        pltpu.VMEM((16,), jnp.uint32),
        pltpu.VMEM((NWIN * 128,), jnp.uint32),
        pltpu.VMEM((NWIN * 128,), jnp.int32),
        pltpu.VMEM((HBITS,), jnp.int32),
        pltpu.VMEM((NSUB, HBITS), jnp.int32),
        pltpu.VMEM((256,), jnp.int32),
        pltpu.VMEM((NWIN, 128), jnp.int32),
        pltpu.SemaphoreType.DMA,
        pltpu.VMEM_SHARED((CAP16 + 16,), jnp.uint32),
        pltpu.VMEM_SHARED((CAP16 + 16,), jnp.int32),
        pltpu.VMEM_SHARED((CAP16 + 16,), jnp.uint32),
        pltpu.VMEM_SHARED((CAP16 + 16,), jnp.int32),
        pltpu.VMEM_SHARED((NSUB, HBITS), jnp.int32),
    ],
)
def _sort_kernel(keys_hbm, tval_hbm, oidx_hbm,
                 chunk_v, tval_v, su_v, si_v, hist_v, allh_v, ctr_v, pos_v,
                 sem, bufA_u, bufA_i, bufB_u, bufB_i, hists_sh):
    w = lax.axis_index("s")
    it16 = _iota16()
    zU = jnp.zeros((16,), jnp.uint32)
    zI = jnp.zeros((16,), jnp.int32)

    pltpu.sync_copy(keys_hbm.at[pl.ds(w * CHUNK16, CHUNK16)], chunk_v)
    pltpu.sync_copy(tval_hbm, tval_v)
    tv = tval_v[...][0]

    def zbody(i, _):
        su_v[pl.ds(i * 16, 16)] = zU
        si_v[pl.ds(i * 16, 16)] = zI
        return 0

    lax.fori_loop(0, NWIN * 8, zbody, 0)

    base = w * CHUNK16

    def cbody(i, cnt):
        u = chunk_v[pl.ds(i * 16, 16)]
        gidx = base + i * 16 + it16
        mask = u >= tv
        plsc.store_compressed(su_v.at[pl.ds(cnt, 16)], u, mask=mask)
        plsc.store_compressed(si_v.at[pl.ds(cnt, 16)], gidx, mask=mask)
        return cnt + jnp.sum(mask.astype(jnp.int32))

    lax.fori_loop(0, CHUNK16_VREGS, cbody, jnp.int32(0))

    rbase = w * PERCAP
    pltpu.sync_copy(su_v.at[pl.ds(0, PERCAP)], bufA_u.at[pl.ds(rbase, PERCAP)])
    pltpu.sync_copy(si_v.at[pl.ds(0, PERCAP)], bufA_i.at[pl.ds(rbase, PERCAP)])
    plsc.subcore_barrier()

    for k in range(2, 8):
        pos_v[NWIN - 1, pl.ds(k * 16, 16)] = CAP16 + it16

    for pno, shift in enumerate((0, 8, 16, 24)):
        src_u, src_i = (bufA_u, bufA_i) if pno % 2 == 0 else (bufB_u, bufB_i)
        dst_u, dst_i = (bufB_u, bufB_i) if pno % 2 == 0 else (bufA_u, bufA_i)

        pltpu.sync_copy(src_u.at[pl.ds(rbase, PERCAP)],
                        su_v.at[pl.ds(0, PERCAP)])
        pltpu.sync_copy(src_i.at[pl.ds(rbase, PERCAP)],
                        si_v.at[pl.ds(0, PERCAP)])
        zeros = jnp.zeros((16,), jnp.int32)
        for j in range(HBITS // 16):
            hist_v[pl.ds(j * 16, 16)] = zeros

        def hbody(i, _):
            u = su_v[pl.ds(i * 16, 16)]
            digit = ((u >> shift) & jnp.uint32(255)).astype(jnp.int32)
            counts, last = plsc.scan_count(digit)
            plsc.addupdate_scatter(hist_v, [digit], counts, mask=last)
            return 0

        lax.fori_loop(0, PERCAP_VREGS, hbody, 0)
        pltpu.sync_copy(hist_v, hists_sh.at[w])
        plsc.subcore_barrier()
        pltpu.sync_copy(hists_sh, allh_v)

        carry = jnp.zeros((16,), jnp.int32)
        for j in range(15, -1, -1):
            tot = jnp.zeros((16,), jnp.int32)
            wsum = jnp.zeros((16,), jnp.int32)
            for w2 in range(NSUB):
                row = allh_v[w2, pl.ds(j * 16, 16)]
                tot = tot + row
                wsum = jnp.where(w2 < w, wsum + row, wsum)
            r = lax.rev(tot, (0,))
            cs = plsc.cumsum(r)
            excl = cs - r
            gbase = carry + lax.rev(excl, (0,))
            ctr_v[pl.ds(j * 16, 16)] = gbase + wsum
            carry = carry + jnp.max(cs)

        def pbody(i, _):
            u = su_v[pl.ds(i * 16, 16)]
            digit = ((u >> shift) & jnp.uint32(255)).astype(jnp.int32)
            counts, last = plsc.scan_count(digit)
            ctr = plsc.load_gather(ctr_v, [digit])
            pos = ctr + counts - 1
            pos_v[i // 8, pl.ds((i % 8) * 16, 16)] = pos
            plsc.addupdate_scatter(ctr_v, [digit], counts, mask=last)
            return 0

        lax.fori_loop(0, PERCAP_VREGS, pbody, 0)

        for g in range(0, NWIN, 8):
            copies = []
            for j in range(g, min(g + 8, NWIN)):
                copies.append(pltpu.async_copy(
                    su_v.at[pl.ds(j * 128, 128)], dst_u.at[pos_v.at[j]], sem))
                copies.append(pltpu.async_copy(
                    si_v.at[pl.ds(j * 128, 128)], dst_i.at[pos_v.at[j]], sem))
            for c in copies:
                c.wait()
        plsc.subcore_barrier()

    pltpu.sync_copy(bufA_i.at[pl.ds(w * O_PER_W16, O_PER_W16)],
                    oidx_hbm.at[pl.ds(w * O_PER_W16, O_PER_W16)])


@functools.partial(
    pl.kernel,
    out_type=(
        jax.ShapeDtypeStruct((OUTPAD,), jnp.float32),
        jax.ShapeDtypeStruct((OUTPAD,), jnp.float32),
    ),
    mesh=_mesh,
    compiler_params=_params,
    scratch_types=[
        pltpu.VMEM((B_PER_W,), jnp.int32),
        pltpu.VMEM((B_PER_W,), jnp.float32),
        pltpu.VMEM((B_PER_W,), jnp.float32),
        pltpu.SemaphoreType.DMA,
        pltpu.SemaphoreType.DMA,
    ],
)
def _gather_kernel(idx_hbm, xs_hbm, ts_hbm, ox_hbm, ot_hbm,
                   idx_v, xrows_v, trows_v, semx, semt):
    w = lax.axis_index("s") * NC + lax.axis_index("c")
    base = w * B_PER_W
    pltpu.sync_copy(idx_hbm.at[pl.ds(base, B_PER_W)], idx_v)
    cx = pltpu.async_copy(xs_hbm.at[idx_v], xrows_v, semx)
    ct = pltpu.async_copy(ts_hbm.at[idx_v], trows_v, semt)
    cx.wait()
    ct.wait()
    pltpu.sync_copy(xrows_v, ox_hbm.at[pl.ds(base, B_PER_W)])
    pltpu.sync_copy(trows_v, ot_hbm.at[pl.ds(base, B_PER_W)])


def _splat16(x, dtype):
    return jnp.broadcast_to(x.astype(dtype), (16,))


def kernel(loss, x_s, t_s):
    w = loss.reshape(-1)
    gkey = jax.random.key(42)
    g = jax.random.gumbel(gkey, w.shape, dtype=w.dtype)
    keys = jnp.log(jnp.maximum(w, 1e-30)) + g
    u_mono = _mono_u32_host(keys)
    keys_p = jnp.concatenate(
        [u_mono, jnp.zeros((N_SP - N_S,), jnp.uint32)])

    pref = jnp.uint32(0)
    n_rem = jnp.int32(N)
    for shift in (24, 16, 8):
        hists = _thresh_hists[shift](keys_p, _splat16(pref, jnp.uint32))
        b, n_rem = _pick_byte(hists, n_rem)
        pref = (pref << 8) | b
    tval = pref << 8

    oidx = _sort_kernel(keys_p, _splat16(tval, jnp.uint32))

    xg, tg = _gather_kernel(oidx, x_s.reshape(-1), t_s.reshape(-1))
    return (xg[:N].reshape(N, 1), tg[:N].reshape(N, 1))


# speedup vs baseline: 27.8282x; 1.8874x over previous
import functools

import jax
import jax.numpy as jnp
from jax import lax
from jax.experimental import pallas as pl
from jax.experimental.pallas import tpu as pltpu
from jax.experimental.pallas import tpu_sc as plsc

N = 100000
N_S = 1000000

_INFO = plsc.get_sparse_core_info()
NC, NSUB, L = _INFO.num_cores, _INFO.num_subcores, _INFO.num_lanes
NW = NC * NSUB

N_SP = 1000448
CHUNK = N_SP // NW
CHUNK_VREGS = CHUNK // L
CHUNK16 = N_SP // NSUB
CHUNK16_VREGS = CHUNK16 // L

PERCAP = 8224
PERCAP_VREGS = PERCAP // L
CAP16 = NSUB * PERCAP
NWIN = (PERCAP + 127) // 128

OUTPAD = 100352
B_PER_W = OUTPAD // NW
O_PER_W16 = OUTPAD // NSUB

HBITS = 272

_mesh = plsc.VectorSubcoreMesh(core_axis_name="c", subcore_axis_name="s")
_params = pltpu.CompilerParams(use_tc_tiling_on_sc=False,
                               needs_layout_passes=False)


def _mono_u32_host(kv):
    b = lax.bitcast_convert_type(kv, jnp.int32)
    m = b ^ ((b >> 31) | jnp.int32(-2147483648))
    return lax.bitcast_convert_type(m, jnp.uint32)


def _iota16():
    return lax.iota(jnp.int32, 16)


def _make_thresh_hist(shift):
    first = shift == 24

    @functools.partial(
        pl.kernel,
        out_type=jax.ShapeDtypeStruct((NW, HBITS), jnp.int32),
        mesh=_mesh,
        compiler_params=_params,
        scratch_types=[
            pltpu.VMEM((CHUNK,), jnp.uint32),
            pltpu.VMEM((16,), jnp.uint32),
            pltpu.VMEM((HBITS,), jnp.int32),
        ],
    )
    def thresh_hist(keys_hbm, pref_hbm, hists_hbm, keys_v, pref_v, hist_v):
        w = lax.axis_index("s") * NC + lax.axis_index("c")
        pltpu.sync_copy(keys_hbm.at[pl.ds(w * CHUNK, CHUNK)], keys_v)
        if not first:
            pltpu.sync_copy(pref_hbm, pref_v)
        zeros = jnp.zeros((16,), jnp.int32)
        for j in range(HBITS // 16):
            hist_v[pl.ds(j * 16, 16)] = zeros
        if first:
            pref = jnp.uint32(0)
        else:
            pref = pref_v[...][0]

        @plsc.parallel_loop(0, CHUNK_VREGS, 1, unroll=4)
        def body(i):
            u = keys_v[pl.ds(i * 16, 16)]
            digit = ((u >> shift) & jnp.uint32(255)).astype(jnp.int32)
            if not first:
                match = (u >> (shift + 8)) == pref
                digit = jnp.where(match, digit, jnp.int32(256))
            counts, last = plsc.scan_count(digit)
            plsc.addupdate_scatter(hist_v, [digit], counts, mask=last)
        pltpu.sync_copy(hist_v, hists_hbm.at[w])

    return thresh_hist


_thresh_hists = {s: _make_thresh_hist(s) for s in (24, 16, 8)}


def _pick_byte(hists, n_rem):
    total = jnp.sum(hists[:, :256], axis=0)
    t = jnp.cumsum(total[::-1])[::-1]
    b = jnp.sum((t >= n_rem).astype(jnp.int32)) - 1
    t_above = jnp.concatenate([t[1:], jnp.zeros((1,), t.dtype)])
    n_rem_next = n_rem - t_above[b]
    return b.astype(jnp.uint32), n_rem_next


@functools.partial(
    pl.kernel,
    out_type=jax.ShapeDtypeStruct((OUTPAD,), jnp.int32),
    mesh=_mesh,
    compiler_params=_params,
    scratch_types=[
        pltpu.VMEM((CHUNK16,), jnp.uint32),
        pltpu.VMEM((16,), jnp.uint32),
        pltpu.VMEM((NWIN * 128,), jnp.uint32),
        pltpu.VMEM((NWIN * 128,), jnp.int32),
        pltpu.VMEM((HBITS,), jnp.int32),
        pltpu.VMEM((NSUB, HBITS), jnp.int32),
        pltpu.VMEM((256,), jnp.int32),
        pltpu.VMEM((NWIN, 128), jnp.int32),
        pltpu.SemaphoreType.DMA,
        pltpu.VMEM_SHARED((CAP16 + 16,), jnp.uint32),
        pltpu.VMEM_SHARED((CAP16 + 16,), jnp.int32),
        pltpu.VMEM_SHARED((CAP16 + 16,), jnp.uint32),
        pltpu.VMEM_SHARED((CAP16 + 16,), jnp.int32),
        pltpu.VMEM_SHARED((NSUB, HBITS), jnp.int32),
    ],
)
def _sort_kernel(keys_hbm, tval_hbm, oidx_hbm,
                 chunk_v, tval_v, su_v, si_v, hist_v, allh_v, ctr_v, pos_v,
                 sem, bufA_u, bufA_i, bufB_u, bufB_i, hists_sh):
    w = lax.axis_index("s")
    it16 = _iota16()
    zU = jnp.zeros((16,), jnp.uint32)
    zI = jnp.zeros((16,), jnp.int32)

    pltpu.sync_copy(keys_hbm.at[pl.ds(w * CHUNK16, CHUNK16)], chunk_v)
    pltpu.sync_copy(tval_hbm, tval_v)
    tv = tval_v[...][0]

    @plsc.parallel_loop(0, NWIN * 8, 1, unroll=8)
    def zbody(i):
        su_v[pl.ds(i * 16, 16)] = zU
        si_v[pl.ds(i * 16, 16)] = zI

    base = w * CHUNK16

    @plsc.parallel_loop(0, CHUNK16_VREGS, 1, unroll=2, carry=jnp.int32(0))
    def cbody(i, cnt):
        u = chunk_v[pl.ds(i * 16, 16)]
        gidx = base + i * 16 + it16
        mask = u >= tv
        plsc.store_compressed(su_v.at[pl.ds(cnt, 16)], u, mask=mask)
        plsc.store_compressed(si_v.at[pl.ds(cnt, 16)], gidx, mask=mask)
        return cnt + jnp.sum(mask.astype(jnp.int32))

    rbase = w * PERCAP
    pltpu.sync_copy(su_v.at[pl.ds(0, PERCAP)], bufA_u.at[pl.ds(rbase, PERCAP)])
    pltpu.sync_copy(si_v.at[pl.ds(0, PERCAP)], bufA_i.at[pl.ds(rbase, PERCAP)])
    plsc.subcore_barrier()

    for k in range(2, 8):
        pos_v[NWIN - 1, pl.ds(k * 16, 16)] = CAP16 + it16

    for pno, shift in enumerate((0, 8, 16, 24)):
        src_u, src_i = (bufA_u, bufA_i) if pno % 2 == 0 else (bufB_u, bufB_i)
        dst_u, dst_i = (bufB_u, bufB_i) if pno % 2 == 0 else (bufA_u, bufA_i)

        pltpu.sync_copy(src_u.at[pl.ds(rbase, PERCAP)],
                        su_v.at[pl.ds(0, PERCAP)])
        pltpu.sync_copy(src_i.at[pl.ds(rbase, PERCAP)],
                        si_v.at[pl.ds(0, PERCAP)])
        zeros = jnp.zeros((16,), jnp.int32)
        for j in range(HBITS // 16):
            hist_v[pl.ds(j * 16, 16)] = zeros

        @plsc.parallel_loop(0, PERCAP_VREGS, 1, unroll=4)
        def hbody(i):
            u = su_v[pl.ds(i * 16, 16)]
            digit = ((u >> shift) & jnp.uint32(255)).astype(jnp.int32)
            counts, last = plsc.scan_count(digit)
            plsc.addupdate_scatter(hist_v, [digit], counts, mask=last)
        pltpu.sync_copy(hist_v, hists_sh.at[w])
        plsc.subcore_barrier()
        pltpu.sync_copy(hists_sh, allh_v)

        carry = jnp.zeros((16,), jnp.int32)
        for j in range(15, -1, -1):
            tot = jnp.zeros((16,), jnp.int32)
            wsum = jnp.zeros((16,), jnp.int32)
            for w2 in range(NSUB):
                row = allh_v[w2, pl.ds(j * 16, 16)]
                tot = tot + row
                wsum = jnp.where(w2 < w, wsum + row, wsum)
            r = lax.rev(tot, (0,))
            cs = plsc.cumsum(r)
            excl = cs - r
            gbase = carry + lax.rev(excl, (0,))
            ctr_v[pl.ds(j * 16, 16)] = gbase + wsum
            carry = carry + jnp.max(cs)

        def pbody(ih, _):
            parts = []
            for k in range(2):
                i = ih * 2 + k
                u = su_v[pl.ds(i * 16, 16)]
                digit = ((u >> shift) & jnp.uint32(255)).astype(jnp.int32)
                counts, last = plsc.scan_count(digit)
                parts.append((i, digit, counts, last))
            for i, digit, counts, last in parts:
                ctr = plsc.load_gather(ctr_v, [digit])
                pos = ctr + counts - 1
                pos_v[i // 8, pl.ds((i % 8) * 16, 16)] = pos
                plsc.addupdate_scatter(ctr_v, [digit], counts, mask=last)
            return 0

        lax.fori_loop(0, PERCAP_VREGS // 2, pbody, 0)

        for g in range(0, NWIN, 8):
            copies = []
            for j in range(g, min(g + 8, NWIN)):
                copies.append(pltpu.async_copy(
                    su_v.at[pl.ds(j * 128, 128)], dst_u.at[pos_v.at[j]], sem))
                copies.append(pltpu.async_copy(
                    si_v.at[pl.ds(j * 128, 128)], dst_i.at[pos_v.at[j]], sem))
            for c in copies:
                c.wait()
        plsc.subcore_barrier()

    pltpu.sync_copy(bufA_i.at[pl.ds(w * O_PER_W16, O_PER_W16)],
                    oidx_hbm.at[pl.ds(w * O_PER_W16, O_PER_W16)])


@functools.partial(
    pl.kernel,
    out_type=(
        jax.ShapeDtypeStruct((OUTPAD,), jnp.float32),
        jax.ShapeDtypeStruct((OUTPAD,), jnp.float32),
    ),
    mesh=_mesh,
    compiler_params=_params,
    scratch_types=[
        pltpu.VMEM((B_PER_W,), jnp.int32),
        pltpu.VMEM((B_PER_W,), jnp.float32),
        pltpu.VMEM((B_PER_W,), jnp.float32),
        pltpu.SemaphoreType.DMA,
        pltpu.SemaphoreType.DMA,
    ],
)
def _gather_kernel(idx_hbm, xs_hbm, ts_hbm, ox_hbm, ot_hbm,
                   idx_v, xrows_v, trows_v, semx, semt):
    w = lax.axis_index("s") * NC + lax.axis_index("c")
    base = w * B_PER_W
    pltpu.sync_copy(idx_hbm.at[pl.ds(base, B_PER_W)], idx_v)
    cx = pltpu.async_copy(xs_hbm.at[idx_v], xrows_v, semx)
    ct = pltpu.async_copy(ts_hbm.at[idx_v], trows_v, semt)
    cx.wait()
    ct.wait()
    pltpu.sync_copy(xrows_v, ox_hbm.at[pl.ds(base, B_PER_W)])
    pltpu.sync_copy(trows_v, ot_hbm.at[pl.ds(base, B_PER_W)])


def _splat16(x, dtype):
    return jnp.broadcast_to(x.astype(dtype), (16,))


def kernel(loss, x_s, t_s):
    w = loss.reshape(-1)
    gkey = jax.random.key(42)
    g = jax.random.gumbel(gkey, w.shape, dtype=w.dtype)
    keys = jnp.log(jnp.maximum(w, 1e-30)) + g
    u_mono = _mono_u32_host(keys)
    keys_p = jnp.concatenate(
        [u_mono, jnp.zeros((N_SP - N_S,), jnp.uint32)])

    pref = jnp.uint32(0)
    n_rem = jnp.int32(N)
    for shift in (24, 16, 8):
        hists = _thresh_hists[shift](keys_p, _splat16(pref, jnp.uint32))
        b, n_rem = _pick_byte(hists, n_rem)
        pref = (pref << 8) | b
    tval = pref << 8

    oidx = _sort_kernel(keys_p, _splat16(tval, jnp.uint32))

    xg, tg = _gather_kernel(oidx, x_s.reshape(-1), t_s.reshape(-1))
    return (xg[:N].reshape(N, 1), tg[:N].reshape(N, 1))


# speedup vs baseline: 32.4016x; 1.1643x over previous
import functools

import jax
import jax.numpy as jnp
from jax import lax
from jax.experimental import pallas as pl
from jax.experimental.pallas import tpu as pltpu
from jax.experimental.pallas import tpu_sc as plsc

N = 100000
N_S = 1000000

_INFO = plsc.get_sparse_core_info()
NC, NSUB, L = _INFO.num_cores, _INFO.num_subcores, _INFO.num_lanes
NW = NC * NSUB

N_SP = 1000448
CHUNK = N_SP // NW
CHUNK_VREGS = CHUNK // L
CHUNK16 = N_SP // NSUB
CHUNK16_VREGS = CHUNK16 // L

PERCAP = 8224
PERCAP_VREGS = PERCAP // L
CAP16 = NSUB * PERCAP
NWIN = (PERCAP + 127) // 128

OUTPAD = 100352
B_PER_W = OUTPAD // NW
O_PER_W16 = OUTPAD // NSUB

HBITS = 272

_mesh = plsc.VectorSubcoreMesh(core_axis_name="c", subcore_axis_name="s")
_params = pltpu.CompilerParams(use_tc_tiling_on_sc=False,
                               needs_layout_passes=False)


def _mono_u32_host(kv):
    b = lax.bitcast_convert_type(kv, jnp.int32)
    m = b ^ ((b >> 31) | jnp.int32(-2147483648))
    return lax.bitcast_convert_type(m, jnp.uint32)


def _iota16():
    return lax.iota(jnp.int32, 16)


def _make_thresh_hist(shift):
    first = shift == 24

    @functools.partial(
        pl.kernel,
        out_type=jax.ShapeDtypeStruct((NW, HBITS), jnp.int32),
        mesh=_mesh,
        compiler_params=_params,
        scratch_types=[
            pltpu.VMEM((CHUNK,), jnp.uint32),
            pltpu.VMEM((16,), jnp.uint32),
            pltpu.VMEM((HBITS,), jnp.int32),
        ],
    )
    def thresh_hist(keys_hbm, pref_hbm, hists_hbm, keys_v, pref_v, hist_v):
        w = lax.axis_index("s") * NC + lax.axis_index("c")
        pltpu.sync_copy(keys_hbm.at[pl.ds(w * CHUNK, CHUNK)], keys_v)
        if not first:
            pltpu.sync_copy(pref_hbm, pref_v)
        zeros = jnp.zeros((16,), jnp.int32)
        for j in range(HBITS // 16):
            hist_v[pl.ds(j * 16, 16)] = zeros
        if first:
            pref = jnp.uint32(0)
        else:
            pref = pref_v[...][0]

        @plsc.parallel_loop(0, CHUNK_VREGS, 1, unroll=4)
        def body(i):
            u = keys_v[pl.ds(i * 16, 16)]
            digit = ((u >> shift) & jnp.uint32(255)).astype(jnp.int32)
            if not first:
                match = (u >> (shift + 8)) == pref
                digit = jnp.where(match, digit, jnp.int32(256))
            counts, last = plsc.scan_count(digit)
            plsc.addupdate_scatter(hist_v, [digit], counts, mask=last)
        pltpu.sync_copy(hist_v, hists_hbm.at[w])

    return thresh_hist


_thresh_hists = {s: _make_thresh_hist(s) for s in (24, 16, 8)}


def _pick_byte(hists, n_rem):
    total = jnp.sum(hists[:, :256], axis=0)
    t = jnp.cumsum(total[::-1])[::-1]
    b = jnp.sum((t >= n_rem).astype(jnp.int32)) - 1
    t_above = jnp.concatenate([t[1:], jnp.zeros((1,), t.dtype)])
    n_rem_next = n_rem - t_above[b]
    return b.astype(jnp.uint32), n_rem_next


@functools.partial(
    pl.kernel,
    out_type=jax.ShapeDtypeStruct((OUTPAD,), jnp.int32),
    mesh=_mesh,
    compiler_params=_params,
    scratch_types=[
        pltpu.VMEM((CHUNK16,), jnp.uint32),
        pltpu.VMEM((NWIN * 128,), jnp.uint32),
        pltpu.VMEM((NWIN * 128,), jnp.int32),
        pltpu.VMEM((HBITS,), jnp.int32),
        pltpu.VMEM((NSUB, HBITS), jnp.int32),
        pltpu.VMEM((256,), jnp.int32),
        pltpu.VMEM((256,), jnp.int32),
        pltpu.VMEM((256,), jnp.int32),
        pltpu.VMEM((NWIN, 128), jnp.int32),
        pltpu.SemaphoreType.DMA,
        pltpu.VMEM_SHARED((CAP16 + 16,), jnp.uint32),
        pltpu.VMEM_SHARED((CAP16 + 16,), jnp.int32),
        pltpu.VMEM_SHARED((CAP16 + 16,), jnp.uint32),
        pltpu.VMEM_SHARED((CAP16 + 16,), jnp.int32),
        pltpu.VMEM_SHARED((NSUB, HBITS), jnp.int32),
    ],
)
def _main_kernel(keys_hbm, oidx_hbm,
                 chunk_v, su_v, si_v, hist_v, allh_v, ctr_v, t_v, tot_v,
                 pos_v, sem,
                 bufA_u, bufA_i, bufB_u, bufB_i, hists_sh):
    w = lax.axis_index("s")
    cid = lax.axis_index("c")
    it16 = _iota16()
    zU = jnp.zeros((16,), jnp.uint32)
    zI = jnp.zeros((16,), jnp.int32)
    zeros = jnp.zeros((16,), jnp.int32)

    pltpu.sync_copy(keys_hbm.at[pl.ds(w * CHUNK16, CHUNK16)], chunk_v)

    prefix = jnp.uint32(0)
    n_rem = jnp.int32(N)
    for tp, shift in enumerate((24, 16, 8)):
        for j in range(HBITS // 16):
            hist_v[pl.ds(j * 16, 16)] = zeros

        @plsc.parallel_loop(0, CHUNK16_VREGS, 1, unroll=4)
        def tbody(i, shift=shift, tp=tp, prefix=prefix):
            u = chunk_v[pl.ds(i * 16, 16)]
            digit = ((u >> shift) & jnp.uint32(255)).astype(jnp.int32)
            if tp > 0:
                match = (u >> (shift + 8)) == prefix
                digit = jnp.where(match, digit, jnp.int32(256))
            counts, last = plsc.scan_count(digit)
            plsc.addupdate_scatter(hist_v, [digit], counts, mask=last)

        pltpu.sync_copy(hist_v, hists_sh.at[w])
        plsc.subcore_barrier()
        pltpu.sync_copy(hists_sh, allh_v)
        plsc.subcore_barrier()

        carry = jnp.int32(0)
        acc = jnp.int32(0)
        for j in range(15, -1, -1):
            tot = jnp.zeros((16,), jnp.int32)
            for w2 in range(NSUB):
                tot = tot + allh_v[w2, pl.ds(j * 16, 16)]
            rt = lax.rev(tot, (0,))
            cs = plsc.cumsum(rt) + carry
            t_v[pl.ds(j * 16, 16)] = lax.rev(cs, (0,))
            tot_v[pl.ds(j * 16, 16)] = tot
            acc = acc + jnp.sum((cs >= n_rem).astype(jnp.int32))
            carry = jnp.max(cs)
        b = acc - 1
        b16 = jnp.broadcast_to(b, (16,))
        tb = jnp.max(plsc.load_gather(t_v, [b16]))
        totb = jnp.max(plsc.load_gather(tot_v, [b16]))
        n_rem = n_rem - (tb - totb)
        prefix = (prefix << 8) | b.astype(jnp.uint32)

    tv = prefix << 8

    @plsc.parallel_loop(0, NWIN * 8, 1, unroll=8)
    def zbody(i):
        su_v[pl.ds(i * 16, 16)] = zU
        si_v[pl.ds(i * 16, 16)] = zI

    base = w * CHUNK16

    @plsc.parallel_loop(0, CHUNK16_VREGS, 1, unroll=2, carry=jnp.int32(0))
    def cbody(i, cnt):
        u = chunk_v[pl.ds(i * 16, 16)]
        gidx = base + i * 16 + it16
        mask = u >= tv
        plsc.store_compressed(su_v.at[pl.ds(cnt, 16)], u, mask=mask)
        plsc.store_compressed(si_v.at[pl.ds(cnt, 16)], gidx, mask=mask)
        return cnt + jnp.sum(mask.astype(jnp.int32))

    rbase = w * PERCAP
    pltpu.sync_copy(su_v.at[pl.ds(0, PERCAP)], bufA_u.at[pl.ds(rbase, PERCAP)])
    pltpu.sync_copy(si_v.at[pl.ds(0, PERCAP)], bufA_i.at[pl.ds(rbase, PERCAP)])
    plsc.subcore_barrier()

    for k in range(2, 8):
        pos_v[NWIN - 1, pl.ds(k * 16, 16)] = CAP16 + it16

    for pno, shift in enumerate((0, 8, 16, 24)):
        src_u, src_i = (bufA_u, bufA_i) if pno % 2 == 0 else (bufB_u, bufB_i)
        dst_u, dst_i = (bufB_u, bufB_i) if pno % 2 == 0 else (bufA_u, bufA_i)

        pltpu.sync_copy(src_u.at[pl.ds(rbase, PERCAP)],
                        su_v.at[pl.ds(0, PERCAP)])
        pltpu.sync_copy(src_i.at[pl.ds(rbase, PERCAP)],
                        si_v.at[pl.ds(0, PERCAP)])
        zeros = jnp.zeros((16,), jnp.int32)
        for j in range(HBITS // 16):
            hist_v[pl.ds(j * 16, 16)] = zeros

        @plsc.parallel_loop(0, PERCAP_VREGS, 1, unroll=4)
        def hbody(i):
            u = su_v[pl.ds(i * 16, 16)]
            digit = ((u >> shift) & jnp.uint32(255)).astype(jnp.int32)
            counts, last = plsc.scan_count(digit)
            plsc.addupdate_scatter(hist_v, [digit], counts, mask=last)
        pltpu.sync_copy(hist_v, hists_sh.at[w])
        plsc.subcore_barrier()
        pltpu.sync_copy(hists_sh, allh_v)

        carry = jnp.zeros((16,), jnp.int32)
        for j in range(15, -1, -1):
            tot = jnp.zeros((16,), jnp.int32)
            wsum = jnp.zeros((16,), jnp.int32)
            for w2 in range(NSUB):
                row = allh_v[w2, pl.ds(j * 16, 16)]
                tot = tot + row
                wsum = jnp.where(w2 < w, wsum + row, wsum)
            r = lax.rev(tot, (0,))
            cs = plsc.cumsum(r)
            excl = cs - r
            gbase = carry + lax.rev(excl, (0,))
            ctr_v[pl.ds(j * 16, 16)] = gbase + wsum
            carry = carry + jnp.max(cs)

        def pbody(ih, _):
            parts = []
            for k in range(2):
                i = ih * 2 + k
                u = su_v[pl.ds(i * 16, 16)]
                digit = ((u >> shift) & jnp.uint32(255)).astype(jnp.int32)
                counts, last = plsc.scan_count(digit)
                parts.append((i, digit, counts, last))
            for i, digit, counts, last in parts:
                ctr = plsc.load_gather(ctr_v, [digit])
                pos = ctr + counts - 1
                pos_v[i // 8, pl.ds((i % 8) * 16, 16)] = pos
                plsc.addupdate_scatter(ctr_v, [digit], counts, mask=last)
            return 0

        lax.fori_loop(0, PERCAP_VREGS // 2, pbody, 0)

        for g in range(0, NWIN, 8):
            copies = []
            for j in range(g, min(g + 8, NWIN)):
                copies.append(pltpu.async_copy(
                    su_v.at[pl.ds(j * 128, 128)], dst_u.at[pos_v.at[j]], sem))
                copies.append(pltpu.async_copy(
                    si_v.at[pl.ds(j * 128, 128)], dst_i.at[pos_v.at[j]], sem))
            for c in copies:
                c.wait()
        plsc.subcore_barrier()

    me = w * NC + cid
    pltpu.sync_copy(bufA_i.at[pl.ds(me * B_PER_W, B_PER_W)],
                    oidx_hbm.at[pl.ds(me * B_PER_W, B_PER_W)])


@functools.partial(
    pl.kernel,
    out_type=(
        jax.ShapeDtypeStruct((OUTPAD,), jnp.float32),
        jax.ShapeDtypeStruct((OUTPAD,), jnp.float32),
    ),
    mesh=_mesh,
    compiler_params=_params,
    scratch_types=[
        pltpu.VMEM((B_PER_W,), jnp.int32),
        pltpu.VMEM((B_PER_W,), jnp.float32),
        pltpu.VMEM((B_PER_W,), jnp.float32),
        pltpu.SemaphoreType.DMA,
        pltpu.SemaphoreType.DMA,
    ],
)
def _gather_kernel(idx_hbm, xs_hbm, ts_hbm, ox_hbm, ot_hbm,
                   idx_v, xrows_v, trows_v, semx, semt):
    w = lax.axis_index("s") * NC + lax.axis_index("c")
    base = w * B_PER_W
    pltpu.sync_copy(idx_hbm.at[pl.ds(base, B_PER_W)], idx_v)
    cx = pltpu.async_copy(xs_hbm.at[idx_v], xrows_v, semx)
    ct = pltpu.async_copy(ts_hbm.at[idx_v], trows_v, semt)
    cx.wait()
    ct.wait()
    pltpu.sync_copy(xrows_v, ox_hbm.at[pl.ds(base, B_PER_W)])
    pltpu.sync_copy(trows_v, ot_hbm.at[pl.ds(base, B_PER_W)])


def _splat16(x, dtype):
    return jnp.broadcast_to(x.astype(dtype), (16,))


def kernel(loss, x_s, t_s):
    w = loss.reshape(-1)
    gkey = jax.random.key(42)
    g = jax.random.gumbel(gkey, w.shape, dtype=w.dtype)
    keys = jnp.log(jnp.maximum(w, 1e-30)) + g
    u_mono = _mono_u32_host(keys)
    keys_p = jnp.concatenate(
        [u_mono, jnp.zeros((N_SP - N_S,), jnp.uint32)])

    oidx = _main_kernel(keys_p)
    xg, tg = _gather_kernel(oidx, x_s.reshape(-1), t_s.reshape(-1))
    return (xg[:N].reshape(N, 1), tg[:N].reshape(N, 1))


# speedup vs baseline: 34.1791x; 1.0549x over previous
import functools

import jax
import jax.numpy as jnp
from jax import lax
from jax.experimental import pallas as pl
from jax.experimental.pallas import tpu as pltpu
from jax.experimental.pallas import tpu_sc as plsc

N = 100000
N_S = 1000000

_INFO = plsc.get_sparse_core_info()
NC, NSUB, L = _INFO.num_cores, _INFO.num_subcores, _INFO.num_lanes
NW = NC * NSUB

N_SP = 1000448
CHUNK = N_SP // NW
CHUNK_VREGS = CHUNK // L
CHUNK16 = N_SP // NSUB
CHUNK16_VREGS = CHUNK16 // L

PERCAP = 8224
PERCAP_VREGS = PERCAP // L
CAP16 = NSUB * PERCAP
NWIN = (PERCAP + 127) // 128

OUTPAD = 100352
B_PER_W = OUTPAD // NW
O_PER_W16 = OUTPAD // NSUB

HBITS = 272

_mesh = plsc.VectorSubcoreMesh(core_axis_name="c", subcore_axis_name="s")
_params = pltpu.CompilerParams(use_tc_tiling_on_sc=False,
                               needs_layout_passes=False)


def _mono_u32_host(kv):
    b = lax.bitcast_convert_type(kv, jnp.int32)
    m = b ^ ((b >> 31) | jnp.int32(-2147483648))
    return lax.bitcast_convert_type(m, jnp.uint32)


def _iota16():
    return lax.iota(jnp.int32, 16)


def _make_thresh_hist(shift):
    first = shift == 24

    @functools.partial(
        pl.kernel,
        out_type=jax.ShapeDtypeStruct((NW, HBITS), jnp.int32),
        mesh=_mesh,
        compiler_params=_params,
        scratch_types=[
            pltpu.VMEM((CHUNK,), jnp.uint32),
            pltpu.VMEM((16,), jnp.uint32),
            pltpu.VMEM((HBITS,), jnp.int32),
        ],
    )
    def thresh_hist(keys_hbm, pref_hbm, hists_hbm, keys_v, pref_v, hist_v):
        w = lax.axis_index("s") * NC + lax.axis_index("c")
        pltpu.sync_copy(keys_hbm.at[pl.ds(w * CHUNK, CHUNK)], keys_v)
        if not first:
            pltpu.sync_copy(pref_hbm, pref_v)
        zeros = jnp.zeros((16,), jnp.int32)
        for j in range(HBITS // 16):
            hist_v[pl.ds(j * 16, 16)] = zeros
        if first:
            pref = jnp.uint32(0)
        else:
            pref = pref_v[...][0]

        @plsc.parallel_loop(0, CHUNK_VREGS, 1, unroll=4)
        def body(i):
            u = keys_v[pl.ds(i * 16, 16)]
            digit = ((u >> shift) & jnp.uint32(255)).astype(jnp.int32)
            if not first:
                match = (u >> (shift + 8)) == pref
                digit = jnp.where(match, digit, jnp.int32(256))
            counts, last = plsc.scan_count(digit)
            plsc.addupdate_scatter(hist_v, [digit], counts, mask=last)
        pltpu.sync_copy(hist_v, hists_hbm.at[w])

    return thresh_hist


_thresh_hists = {s: _make_thresh_hist(s) for s in (24, 16, 8)}


def _pick_byte(hists, n_rem):
    total = jnp.sum(hists[:, :256], axis=0)
    t = jnp.cumsum(total[::-1])[::-1]
    b = jnp.sum((t >= n_rem).astype(jnp.int32)) - 1
    t_above = jnp.concatenate([t[1:], jnp.zeros((1,), t.dtype)])
    n_rem_next = n_rem - t_above[b]
    return b.astype(jnp.uint32), n_rem_next


@functools.partial(
    pl.kernel,
    out_type=jax.ShapeDtypeStruct((OUTPAD,), jnp.int32),
    mesh=_mesh,
    compiler_params=_params,
    scratch_types=[
        pltpu.VMEM((CHUNK16,), jnp.uint32),
        pltpu.VMEM((NWIN * 128,), jnp.uint32),
        pltpu.VMEM((NWIN * 128,), jnp.int32),
        pltpu.VMEM((HBITS,), jnp.int32),
        pltpu.VMEM((NSUB, HBITS), jnp.int32),
        pltpu.VMEM((256,), jnp.int32),
        pltpu.VMEM((256,), jnp.int32),
        pltpu.VMEM((256,), jnp.int32),
        pltpu.VMEM((NWIN, 128), jnp.int32),
        pltpu.SemaphoreType.DMA,
        pltpu.VMEM_SHARED((CAP16 + 16,), jnp.uint32),
        pltpu.VMEM_SHARED((CAP16 + 16,), jnp.int32),
        pltpu.VMEM_SHARED((CAP16 + 16,), jnp.uint32),
        pltpu.VMEM_SHARED((CAP16 + 16,), jnp.int32),
        pltpu.VMEM_SHARED((NSUB, HBITS), jnp.int32),
    ],
)
def _main_kernel(keys_hbm, oidx_hbm,
                 chunk_v, su_v, si_v, hist_v, allh_v, ctr_v, t_v, tot_v,
                 pos_v, sem,
                 bufA_u, bufA_i, bufB_u, bufB_i, hists_sh):
    w = lax.axis_index("s")
    cid = lax.axis_index("c")
    it16 = _iota16()
    zU = jnp.zeros((16,), jnp.uint32)
    zI = jnp.zeros((16,), jnp.int32)
    zeros = jnp.zeros((16,), jnp.int32)

    pltpu.sync_copy(keys_hbm.at[pl.ds(w * CHUNK16, CHUNK16)], chunk_v)

    prefix = jnp.uint32(0)
    n_rem = jnp.int32(N)
    for tp, shift in enumerate((24, 16)):
        for j in range(HBITS // 16):
            hist_v[pl.ds(j * 16, 16)] = zeros

        @plsc.parallel_loop(0, CHUNK16_VREGS, 1, unroll=4)
        def tbody(i, shift=shift, tp=tp, prefix=prefix):
            u = chunk_v[pl.ds(i * 16, 16)]
            digit = ((u >> shift) & jnp.uint32(255)).astype(jnp.int32)
            if tp > 0:
                match = (u >> (shift + 8)) == prefix
                digit = jnp.where(match, digit, jnp.int32(256))
            counts, last = plsc.scan_count(digit)
            plsc.addupdate_scatter(hist_v, [digit], counts, mask=last)

        pltpu.sync_copy(hist_v, hists_sh.at[w])
        plsc.subcore_barrier()
        pltpu.sync_copy(hists_sh, allh_v)
        plsc.subcore_barrier()

        carry = jnp.int32(0)
        acc = jnp.int32(0)
        for j in range(15, -1, -1):
            tot = jnp.zeros((16,), jnp.int32)
            for w2 in range(NSUB):
                tot = tot + allh_v[w2, pl.ds(j * 16, 16)]
            rt = lax.rev(tot, (0,))
            cs = plsc.cumsum(rt) + carry
            t_v[pl.ds(j * 16, 16)] = lax.rev(cs, (0,))
            tot_v[pl.ds(j * 16, 16)] = tot
            acc = acc + jnp.sum((cs >= n_rem).astype(jnp.int32))
            carry = jnp.max(cs)
        b = acc - 1
        b16 = jnp.broadcast_to(b, (16,))
        tb = jnp.max(plsc.load_gather(t_v, [b16]))
        totb = jnp.max(plsc.load_gather(tot_v, [b16]))
        n_rem = n_rem - (tb - totb)
        prefix = (prefix << 8) | b.astype(jnp.uint32)

    tv = prefix << 16

    @plsc.parallel_loop(0, NWIN * 8, 1, unroll=8)
    def zbody(i):
        su_v[pl.ds(i * 16, 16)] = zU
        si_v[pl.ds(i * 16, 16)] = zI

    base = w * CHUNK16

    @plsc.parallel_loop(0, CHUNK16_VREGS, 1, unroll=4, carry=jnp.int32(0))
    def cbody(i, cnt):
        u = chunk_v[pl.ds(i * 16, 16)]
        gidx = base + i * 16 + it16
        mask = u >= tv
        plsc.store_compressed(su_v.at[pl.ds(cnt, 16)], u, mask=mask)
        plsc.store_compressed(si_v.at[pl.ds(cnt, 16)], gidx, mask=mask)
        return cnt + jnp.sum(mask.astype(jnp.int32))

    rbase = w * PERCAP
    pltpu.sync_copy(su_v.at[pl.ds(0, PERCAP)], bufA_u.at[pl.ds(rbase, PERCAP)])
    pltpu.sync_copy(si_v.at[pl.ds(0, PERCAP)], bufA_i.at[pl.ds(rbase, PERCAP)])
    plsc.subcore_barrier()

    for k in range(2, 8):
        pos_v[NWIN - 1, pl.ds(k * 16, 16)] = CAP16 + it16

    for pno, shift in enumerate((0, 8, 16, 24)):
        src_u, src_i = (bufA_u, bufA_i) if pno % 2 == 0 else (bufB_u, bufB_i)
        dst_u, dst_i = (bufB_u, bufB_i) if pno % 2 == 0 else (bufA_u, bufA_i)

        pltpu.sync_copy(src_u.at[pl.ds(rbase, PERCAP)],
                        su_v.at[pl.ds(0, PERCAP)])
        pltpu.sync_copy(src_i.at[pl.ds(rbase, PERCAP)],
                        si_v.at[pl.ds(0, PERCAP)])
        zeros = jnp.zeros((16,), jnp.int32)
        for j in range(HBITS // 16):
            hist_v[pl.ds(j * 16, 16)] = zeros

        @plsc.parallel_loop(0, PERCAP_VREGS, 1, unroll=4)
        def hbody(i):
            u = su_v[pl.ds(i * 16, 16)]
            digit = ((u >> shift) & jnp.uint32(255)).astype(jnp.int32)
            counts, last = plsc.scan_count(digit)
            plsc.addupdate_scatter(hist_v, [digit], counts, mask=last)
        pltpu.sync_copy(hist_v, hists_sh.at[w])
        plsc.subcore_barrier()
        pltpu.sync_copy(hists_sh, allh_v)

        carry = jnp.zeros((16,), jnp.int32)
        for j in range(15, -1, -1):
            tot = jnp.zeros((16,), jnp.int32)
            wsum = jnp.zeros((16,), jnp.int32)
            for w2 in range(NSUB):
                row = allh_v[w2, pl.ds(j * 16, 16)]
                tot = tot + row
                wsum = jnp.where(w2 < w, wsum + row, wsum)
            r = lax.rev(tot, (0,))
            cs = plsc.cumsum(r)
            excl = cs - r
            gbase = carry + lax.rev(excl, (0,))
            ctr_v[pl.ds(j * 16, 16)] = gbase + wsum
            carry = carry + jnp.max(cs)

        def pbody(ih, _):
            parts = []
            for k in range(2):
                i = ih * 2 + k
                u = su_v[pl.ds(i * 16, 16)]
                digit = ((u >> shift) & jnp.uint32(255)).astype(jnp.int32)
                counts, last = plsc.scan_count(digit)
                parts.append((i, digit, counts, last))
            for i, digit, counts, last in parts:
                ctr = plsc.load_gather(ctr_v, [digit])
                pos = ctr + counts - 1
                pos_v[i // 8, pl.ds((i % 8) * 16, 16)] = pos
                plsc.addupdate_scatter(ctr_v, [digit], counts, mask=last)
            return 0

        lax.fori_loop(0, PERCAP_VREGS // 2, pbody, 0)

        for g in range(0, NWIN, 8):
            copies = []
            for j in range(g, min(g + 8, NWIN)):
                copies.append(pltpu.async_copy(
                    su_v.at[pl.ds(j * 128, 128)], dst_u.at[pos_v.at[j]], sem))
                copies.append(pltpu.async_copy(
                    si_v.at[pl.ds(j * 128, 128)], dst_i.at[pos_v.at[j]], sem))
            for c in copies:
                c.wait()
        plsc.subcore_barrier()

    me = w * NC + cid
    pltpu.sync_copy(bufA_i.at[pl.ds(me * B_PER_W, B_PER_W)],
                    oidx_hbm.at[pl.ds(me * B_PER_W, B_PER_W)])


@functools.partial(
    pl.kernel,
    out_type=(
        jax.ShapeDtypeStruct((OUTPAD,), jnp.float32),
        jax.ShapeDtypeStruct((OUTPAD,), jnp.float32),
    ),
    mesh=_mesh,
    compiler_params=_params,
    scratch_types=[
        pltpu.VMEM((B_PER_W,), jnp.int32),
        pltpu.VMEM((B_PER_W,), jnp.float32),
        pltpu.VMEM((B_PER_W,), jnp.float32),
        pltpu.SemaphoreType.DMA,
        pltpu.SemaphoreType.DMA,
    ],
)
def _gather_kernel(idx_hbm, xs_hbm, ts_hbm, ox_hbm, ot_hbm,
                   idx_v, xrows_v, trows_v, semx, semt):
    w = lax.axis_index("s") * NC + lax.axis_index("c")
    base = w * B_PER_W
    pltpu.sync_copy(idx_hbm.at[pl.ds(base, B_PER_W)], idx_v)
    cx = pltpu.async_copy(xs_hbm.at[idx_v], xrows_v, semx)
    ct = pltpu.async_copy(ts_hbm.at[idx_v], trows_v, semt)
    cx.wait()
    ct.wait()
    pltpu.sync_copy(xrows_v, ox_hbm.at[pl.ds(base, B_PER_W)])
    pltpu.sync_copy(trows_v, ot_hbm.at[pl.ds(base, B_PER_W)])


def _splat16(x, dtype):
    return jnp.broadcast_to(x.astype(dtype), (16,))


def kernel(loss, x_s, t_s):
    w = loss.reshape(-1)
    gkey = jax.random.key(42)
    g = jax.random.gumbel(gkey, w.shape, dtype=w.dtype)
    keys = jnp.log(jnp.maximum(w, 1e-30)) + g
    u_mono = _mono_u32_host(keys)
    keys_p = jnp.concatenate(
        [u_mono, jnp.zeros((N_SP - N_S,), jnp.uint32)])

    oidx = _main_kernel(keys_p)
    xg, tg = _gather_kernel(oidx, x_s.reshape(-1), t_s.reshape(-1))
    return (xg[:N].reshape(N, 1), tg[:N].reshape(N, 1))


# speedup vs baseline: 34.3252x; 1.0043x over previous
import functools

import jax
import jax.numpy as jnp
from jax import lax
from jax.experimental import pallas as pl
from jax.experimental.pallas import tpu as pltpu
from jax.experimental.pallas import tpu_sc as plsc

N = 100000
N_S = 1000000

_INFO = plsc.get_sparse_core_info()
NC, NSUB, L = _INFO.num_cores, _INFO.num_subcores, _INFO.num_lanes
NW = NC * NSUB

N_SP = 1000448
CHUNK = N_SP // NW
CHUNK_VREGS = CHUNK // L
CHUNK16 = N_SP // NSUB
CHUNK16_VREGS = CHUNK16 // L

PERCAP = 8224
PERCAP_VREGS = PERCAP // L
CAP16 = NSUB * PERCAP
NWIN = (PERCAP + 127) // 128

OUTPAD = 100352
B_PER_W = OUTPAD // NW
O_PER_W16 = OUTPAD // NSUB

HBITS = 272

_mesh = plsc.VectorSubcoreMesh(core_axis_name="c", subcore_axis_name="s")
_params = pltpu.CompilerParams(use_tc_tiling_on_sc=False,
                               needs_layout_passes=False)


def _mono_u32_host(kv):
    b = lax.bitcast_convert_type(kv, jnp.int32)
    m = b ^ ((b >> 31) | jnp.int32(-2147483648))
    return lax.bitcast_convert_type(m, jnp.uint32)


def _iota16():
    return lax.iota(jnp.int32, 16)


def _make_thresh_hist(shift):
    first = shift == 24

    @functools.partial(
        pl.kernel,
        out_type=jax.ShapeDtypeStruct((NW, HBITS), jnp.int32),
        mesh=_mesh,
        compiler_params=_params,
        scratch_types=[
            pltpu.VMEM((CHUNK,), jnp.uint32),
            pltpu.VMEM((16,), jnp.uint32),
            pltpu.VMEM((HBITS,), jnp.int32),
        ],
    )
    def thresh_hist(keys_hbm, pref_hbm, hists_hbm, keys_v, pref_v, hist_v):
        w = lax.axis_index("s") * NC + lax.axis_index("c")
        pltpu.sync_copy(keys_hbm.at[pl.ds(w * CHUNK, CHUNK)], keys_v)
        if not first:
            pltpu.sync_copy(pref_hbm, pref_v)
        zeros = jnp.zeros((16,), jnp.int32)
        for j in range(HBITS // 16):
            hist_v[pl.ds(j * 16, 16)] = zeros
        if first:
            pref = jnp.uint32(0)
        else:
            pref = pref_v[...][0]

        @plsc.parallel_loop(0, CHUNK_VREGS, 1, unroll=4)
        def body(i):
            u = keys_v[pl.ds(i * 16, 16)]
            digit = ((u >> shift) & jnp.uint32(255)).astype(jnp.int32)
            if not first:
                match = (u >> (shift + 8)) == pref
                digit = jnp.where(match, digit, jnp.int32(256))
            counts, last = plsc.scan_count(digit)
            plsc.addupdate_scatter(hist_v, [digit], counts, mask=last)
        pltpu.sync_copy(hist_v, hists_hbm.at[w])

    return thresh_hist


_thresh_hists = {s: _make_thresh_hist(s) for s in (24, 16, 8)}


def _pick_byte(hists, n_rem):
    total = jnp.sum(hists[:, :256], axis=0)
    t = jnp.cumsum(total[::-1])[::-1]
    b = jnp.sum((t >= n_rem).astype(jnp.int32)) - 1
    t_above = jnp.concatenate([t[1:], jnp.zeros((1,), t.dtype)])
    n_rem_next = n_rem - t_above[b]
    return b.astype(jnp.uint32), n_rem_next


@functools.partial(
    pl.kernel,
    out_type=jax.ShapeDtypeStruct((OUTPAD,), jnp.int32),
    mesh=_mesh,
    compiler_params=_params,
    scratch_types=[
        pltpu.VMEM((CHUNK16,), jnp.uint32),
        pltpu.VMEM((NWIN * 128,), jnp.uint32),
        pltpu.VMEM((NWIN * 128,), jnp.int32),
        pltpu.VMEM((HBITS,), jnp.int32),
        pltpu.VMEM((NSUB, HBITS), jnp.int32),
        pltpu.VMEM((256,), jnp.int32),
        pltpu.VMEM((256,), jnp.int32),
        pltpu.VMEM((256,), jnp.int32),
        pltpu.VMEM((NWIN, 128), jnp.int32),
        pltpu.SemaphoreType.DMA,
        pltpu.VMEM_SHARED((CAP16 + 16,), jnp.uint32),
        pltpu.VMEM_SHARED((CAP16 + 16,), jnp.int32),
        pltpu.VMEM_SHARED((CAP16 + 16,), jnp.uint32),
        pltpu.VMEM_SHARED((CAP16 + 16,), jnp.int32),
        pltpu.VMEM_SHARED((NSUB, HBITS), jnp.int32),
    ],
)
def _main_kernel(keys_hbm, oidx_hbm,
                 chunk_v, su_v, si_v, hist_v, allh_v, ctr_v, t_v, tot_v,
                 pos_v, sem,
                 bufA_u, bufA_i, bufB_u, bufB_i, hists_sh):
    w = lax.axis_index("s")
    cid = lax.axis_index("c")
    it16 = _iota16()
    zU = jnp.zeros((16,), jnp.uint32)
    zI = jnp.zeros((16,), jnp.int32)
    zeros = jnp.zeros((16,), jnp.int32)

    pltpu.sync_copy(keys_hbm.at[pl.ds(w * CHUNK16, CHUNK16)], chunk_v)

    prefix = jnp.uint32(0)
    n_rem = jnp.int32(N)
    for tp, shift in enumerate((24, 16)):
        for j in range(HBITS // 16):
            hist_v[pl.ds(j * 16, 16)] = zeros

        @plsc.parallel_loop(0, CHUNK16_VREGS, 1, unroll=6)
        def tbody(i, shift=shift, tp=tp, prefix=prefix):
            u = chunk_v[pl.ds(i * 16, 16)]
            digit = ((u >> shift) & jnp.uint32(255)).astype(jnp.int32)
            if tp > 0:
                match = (u >> (shift + 8)) == prefix
                digit = jnp.where(match, digit, jnp.int32(256))
            counts, last = plsc.scan_count(digit)
            plsc.addupdate_scatter(hist_v, [digit], counts, mask=last)

        pltpu.sync_copy(hist_v, hists_sh.at[w])
        plsc.subcore_barrier()
        pltpu.sync_copy(hists_sh, allh_v)
        plsc.subcore_barrier()

        carry = jnp.int32(0)
        acc = jnp.int32(0)
        for j in range(15, -1, -1):
            tot = jnp.zeros((16,), jnp.int32)
            for w2 in range(NSUB):
                tot = tot + allh_v[w2, pl.ds(j * 16, 16)]
            rt = lax.rev(tot, (0,))
            cs = plsc.cumsum(rt) + carry
            t_v[pl.ds(j * 16, 16)] = lax.rev(cs, (0,))
            tot_v[pl.ds(j * 16, 16)] = tot
            acc = acc + jnp.sum((cs >= n_rem).astype(jnp.int32))
            carry = jnp.max(cs)
        b = acc - 1
        b16 = jnp.broadcast_to(b, (16,))
        tb = jnp.max(plsc.load_gather(t_v, [b16]))
        totb = jnp.max(plsc.load_gather(tot_v, [b16]))
        n_rem = n_rem - (tb - totb)
        prefix = (prefix << 8) | b.astype(jnp.uint32)

    tv = prefix << 16

    @plsc.parallel_loop(0, NWIN * 8, 1, unroll=8)
    def zbody(i):
        su_v[pl.ds(i * 16, 16)] = zU
        si_v[pl.ds(i * 16, 16)] = zI

    base = w * CHUNK16

    @plsc.parallel_loop(0, CHUNK16_VREGS, 1, unroll=4, carry=jnp.int32(0))
    def cbody(i, cnt):
        u = chunk_v[pl.ds(i * 16, 16)]
        gidx = base + i * 16 + it16
        mask = u >= tv
        plsc.store_compressed(su_v.at[pl.ds(cnt, 16)], u, mask=mask)
        plsc.store_compressed(si_v.at[pl.ds(cnt, 16)], gidx, mask=mask)
        return cnt + jnp.sum(mask.astype(jnp.int32))

    rbase = w * PERCAP
    pltpu.sync_copy(su_v.at[pl.ds(0, PERCAP)], bufA_u.at[pl.ds(rbase, PERCAP)])
    pltpu.sync_copy(si_v.at[pl.ds(0, PERCAP)], bufA_i.at[pl.ds(rbase, PERCAP)])
    plsc.subcore_barrier()

    for k in range(2, 8):
        pos_v[NWIN - 1, pl.ds(k * 16, 16)] = CAP16 + it16

    for pno, shift in enumerate((0, 8, 16, 24)):
        src_u, src_i = (bufA_u, bufA_i) if pno % 2 == 0 else (bufB_u, bufB_i)
        dst_u, dst_i = (bufB_u, bufB_i) if pno % 2 == 0 else (bufA_u, bufA_i)

        pltpu.sync_copy(src_u.at[pl.ds(rbase, PERCAP)],
                        su_v.at[pl.ds(0, PERCAP)])
        pltpu.sync_copy(src_i.at[pl.ds(rbase, PERCAP)],
                        si_v.at[pl.ds(0, PERCAP)])
        zeros = jnp.zeros((16,), jnp.int32)
        for j in range(HBITS // 16):
            hist_v[pl.ds(j * 16, 16)] = zeros

        @plsc.parallel_loop(0, PERCAP_VREGS, 1, unroll=6)
        def hbody(i):
            u = su_v[pl.ds(i * 16, 16)]
            digit = ((u >> shift) & jnp.uint32(255)).astype(jnp.int32)
            counts, last = plsc.scan_count(digit)
            plsc.addupdate_scatter(hist_v, [digit], counts, mask=last)
        pltpu.sync_copy(hist_v, hists_sh.at[w])
        plsc.subcore_barrier()
        pltpu.sync_copy(hists_sh, allh_v)

        carry = jnp.zeros((16,), jnp.int32)
        for j in range(15, -1, -1):
            tot = jnp.zeros((16,), jnp.int32)
            wsum = jnp.zeros((16,), jnp.int32)
            for w2 in range(NSUB):
                row = allh_v[w2, pl.ds(j * 16, 16)]
                tot = tot + row
                wsum = jnp.where(w2 < w, wsum + row, wsum)
            r = lax.rev(tot, (0,))
            cs = plsc.cumsum(r)
            excl = cs - r
            gbase = carry + lax.rev(excl, (0,))
            ctr_v[pl.ds(j * 16, 16)] = gbase + wsum
            carry = carry + jnp.max(cs)

        def pbody(ih, _):
            parts = []
            for k in range(2):
                i = ih * 2 + k
                u = su_v[pl.ds(i * 16, 16)]
                digit = ((u >> shift) & jnp.uint32(255)).astype(jnp.int32)
                counts, last = plsc.scan_count(digit)
                parts.append((i, digit, counts, last))
            for i, digit, counts, last in parts:
                ctr = plsc.load_gather(ctr_v, [digit])
                pos = ctr + counts - 1
                pos_v[i // 8, pl.ds((i % 8) * 16, 16)] = pos
                plsc.addupdate_scatter(ctr_v, [digit], counts, mask=last)
            return 0

        lax.fori_loop(0, PERCAP_VREGS // 2, pbody, 0)

        for g in range(0, NWIN, 16):
            copies = []
            for j in range(g, min(g + 16, NWIN)):
                copies.append(pltpu.async_copy(
                    su_v.at[pl.ds(j * 128, 128)], dst_u.at[pos_v.at[j]], sem))
                copies.append(pltpu.async_copy(
                    si_v.at[pl.ds(j * 128, 128)], dst_i.at[pos_v.at[j]], sem))
            for c in copies:
                c.wait()
        plsc.subcore_barrier()

    me = w * NC + cid
    pltpu.sync_copy(bufA_i.at[pl.ds(me * B_PER_W, B_PER_W)],
                    oidx_hbm.at[pl.ds(me * B_PER_W, B_PER_W)])


@functools.partial(
    pl.kernel,
    out_type=(
        jax.ShapeDtypeStruct((OUTPAD,), jnp.float32),
        jax.ShapeDtypeStruct((OUTPAD,), jnp.float32),
    ),
    mesh=_mesh,
    compiler_params=_params,
    scratch_types=[
        pltpu.VMEM((B_PER_W,), jnp.int32),
        pltpu.VMEM((B_PER_W,), jnp.float32),
        pltpu.VMEM((B_PER_W,), jnp.float32),
        pltpu.SemaphoreType.DMA,
        pltpu.SemaphoreType.DMA,
    ],
)
def _gather_kernel(idx_hbm, xs_hbm, ts_hbm, ox_hbm, ot_hbm,
                   idx_v, xrows_v, trows_v, semx, semt):
    w = lax.axis_index("s") * NC + lax.axis_index("c")
    base = w * B_PER_W
    pltpu.sync_copy(idx_hbm.at[pl.ds(base, B_PER_W)], idx_v)
    cx = pltpu.async_copy(xs_hbm.at[idx_v], xrows_v, semx)
    ct = pltpu.async_copy(ts_hbm.at[idx_v], trows_v, semt)
    cx.wait()
    ct.wait()
    pltpu.sync_copy(xrows_v, ox_hbm.at[pl.ds(base, B_PER_W)])
    pltpu.sync_copy(trows_v, ot_hbm.at[pl.ds(base, B_PER_W)])


def _splat16(x, dtype):
    return jnp.broadcast_to(x.astype(dtype), (16,))


def kernel(loss, x_s, t_s):
    w = loss.reshape(-1)
    gkey = jax.random.key(42)
    g = jax.random.gumbel(gkey, w.shape, dtype=w.dtype)
    keys = jnp.log(jnp.maximum(w, 1e-30)) + g
    u_mono = _mono_u32_host(keys)
    keys_p = jnp.concatenate(
        [u_mono, jnp.zeros((N_SP - N_S,), jnp.uint32)])

    oidx = _main_kernel(keys_p)
    xg, tg = _gather_kernel(oidx, x_s.reshape(-1), t_s.reshape(-1))
    return (xg[:N].reshape(N, 1), tg[:N].reshape(N, 1))


# speedup vs baseline: 34.3287x; 1.0001x over previous
import functools

import jax
import jax.numpy as jnp
from jax import lax
from jax.experimental import pallas as pl
from jax.experimental.pallas import tpu as pltpu
from jax.experimental.pallas import tpu_sc as plsc

N = 100000
N_S = 1000000

_INFO = plsc.get_sparse_core_info()
NC, NSUB, L = _INFO.num_cores, _INFO.num_subcores, _INFO.num_lanes
NW = NC * NSUB

N_SP = 1000448
CHUNK = N_SP // NW
CHUNK_VREGS = CHUNK // L
CHUNK16 = N_SP // NSUB
CHUNK16_VREGS = CHUNK16 // L

PERCAP = 8224
PERCAP_VREGS = PERCAP // L
CAP16 = NSUB * PERCAP
NWIN = (PERCAP + 127) // 128

OUTPAD = 100352
B_PER_W = OUTPAD // NW

HBITS = 272

_mesh = plsc.VectorSubcoreMesh(core_axis_name="c", subcore_axis_name="s")
_params = pltpu.CompilerParams(use_tc_tiling_on_sc=False,
                               needs_layout_passes=False)


def _mono_u32_host(kv):
    b = lax.bitcast_convert_type(kv, jnp.int32)
    m = b ^ ((b >> 31) | jnp.int32(-2147483648))
    return lax.bitcast_convert_type(m, jnp.uint32)


def _iota16():
    return lax.iota(jnp.int32, 16)


@functools.partial(
    pl.kernel,
    out_type=jax.ShapeDtypeStruct((OUTPAD,), jnp.int32),
    mesh=_mesh,
    compiler_params=_params,
    scratch_types=[
        pltpu.VMEM((CHUNK16,), jnp.uint32),
        pltpu.VMEM((NWIN * 128,), jnp.uint32),
        pltpu.VMEM((NWIN * 128,), jnp.int32),
        pltpu.VMEM((HBITS,), jnp.int32),
        pltpu.VMEM((NSUB, HBITS), jnp.int32),
        pltpu.VMEM((256,), jnp.int32),
        pltpu.VMEM((256,), jnp.int32),
        pltpu.VMEM((256,), jnp.int32),
        pltpu.VMEM((NWIN, 128), jnp.int32),
        pltpu.SemaphoreType.DMA,
        pltpu.VMEM_SHARED((CAP16 + 16,), jnp.uint32),
        pltpu.VMEM_SHARED((CAP16 + 16,), jnp.int32),
        pltpu.VMEM_SHARED((CAP16 + 16,), jnp.uint32),
        pltpu.VMEM_SHARED((CAP16 + 16,), jnp.int32),
        pltpu.VMEM_SHARED((NSUB, HBITS), jnp.int32),
    ],
)
def _main_kernel(keys_hbm, oidx_hbm,
                 chunk_v, su_v, si_v, hist_v, allh_v, ctr_v, t_v, tot_v,
                 pos_v, sem,
                 bufA_u, bufA_i, bufB_u, bufB_i, hists_sh):
    w = lax.axis_index("s")
    cid = lax.axis_index("c")
    it16 = _iota16()
    zU = jnp.zeros((16,), jnp.uint32)
    zI = jnp.zeros((16,), jnp.int32)
    zeros = jnp.zeros((16,), jnp.int32)

    pltpu.sync_copy(keys_hbm.at[pl.ds(w * CHUNK16, CHUNK16)], chunk_v)

    prefix = jnp.uint32(0)
    n_rem = jnp.int32(N)
    for tp, shift in enumerate((24, 16)):
        for j in range(HBITS // 16):
            hist_v[pl.ds(j * 16, 16)] = zeros

        @plsc.parallel_loop(0, CHUNK16_VREGS, 1, unroll=6)
        def tbody(i, shift=shift, tp=tp, prefix=prefix):
            u = chunk_v[pl.ds(i * 16, 16)]
            digit = ((u >> shift) & jnp.uint32(255)).astype(jnp.int32)
            if tp > 0:
                match = (u >> (shift + 8)) == prefix
                digit = jnp.where(match, digit, jnp.int32(256))
            counts, last = plsc.scan_count(digit)
            plsc.addupdate_scatter(hist_v, [digit], counts, mask=last)

        pltpu.sync_copy(hist_v, hists_sh.at[w])
        plsc.subcore_barrier()
        pltpu.sync_copy(hists_sh, allh_v)
        plsc.subcore_barrier()

        carry = jnp.int32(0)
        acc = jnp.int32(0)
        for j in range(15, -1, -1):
            tot = jnp.zeros((16,), jnp.int32)
            for w2 in range(NSUB):
                tot = tot + allh_v[w2, pl.ds(j * 16, 16)]
            rt = lax.rev(tot, (0,))
            cs = plsc.cumsum(rt) + carry
            t_v[pl.ds(j * 16, 16)] = lax.rev(cs, (0,))
            tot_v[pl.ds(j * 16, 16)] = tot
            acc = acc + jnp.sum((cs >= n_rem).astype(jnp.int32))
            carry = jnp.max(cs)
        b = acc - 1
        b16 = jnp.broadcast_to(b, (16,))
        tb = jnp.max(plsc.load_gather(t_v, [b16]))
        totb = jnp.max(plsc.load_gather(tot_v, [b16]))
        n_rem = n_rem - (tb - totb)
        prefix = (prefix << 8) | b.astype(jnp.uint32)

    tv = prefix << 16

    @plsc.parallel_loop(0, NWIN * 8, 1, unroll=8)
    def zbody(i):
        su_v[pl.ds(i * 16, 16)] = zU
        si_v[pl.ds(i * 16, 16)] = zI

    base = w * CHUNK16

    @plsc.parallel_loop(0, CHUNK16_VREGS, 1, unroll=4, carry=jnp.int32(0))
    def cbody(i, cnt):
        u = chunk_v[pl.ds(i * 16, 16)]
        gidx = base + i * 16 + it16
        mask = u >= tv
        plsc.store_compressed(su_v.at[pl.ds(cnt, 16)], u, mask=mask)
        plsc.store_compressed(si_v.at[pl.ds(cnt, 16)], gidx, mask=mask)
        return cnt + jnp.sum(mask.astype(jnp.int32))

    rbase = w * PERCAP
    pltpu.sync_copy(su_v.at[pl.ds(0, PERCAP)], bufA_u.at[pl.ds(rbase, PERCAP)])
    pltpu.sync_copy(si_v.at[pl.ds(0, PERCAP)], bufA_i.at[pl.ds(rbase, PERCAP)])
    plsc.subcore_barrier()

    for k in range(2, 8):
        pos_v[NWIN - 1, pl.ds(k * 16, 16)] = CAP16 + it16

    for pno, shift in enumerate((0, 8, 16, 24)):
        src_u, src_i = (bufA_u, bufA_i) if pno % 2 == 0 else (bufB_u, bufB_i)
        dst_u, dst_i = (bufB_u, bufB_i) if pno % 2 == 0 else (bufA_u, bufA_i)

        pltpu.sync_copy(src_u.at[pl.ds(rbase, PERCAP)],
                        su_v.at[pl.ds(0, PERCAP)])
        pltpu.sync_copy(src_i.at[pl.ds(rbase, PERCAP)],
                        si_v.at[pl.ds(0, PERCAP)])
        zeros = jnp.zeros((16,), jnp.int32)
        for j in range(HBITS // 16):
            hist_v[pl.ds(j * 16, 16)] = zeros

        @plsc.parallel_loop(0, PERCAP_VREGS, 1, unroll=6)
        def hbody(i):
            u = su_v[pl.ds(i * 16, 16)]
            digit = ((u >> shift) & jnp.uint32(255)).astype(jnp.int32)
            counts, last = plsc.scan_count(digit)
            plsc.addupdate_scatter(hist_v, [digit], counts, mask=last)
        pltpu.sync_copy(hist_v, hists_sh.at[w])
        plsc.subcore_barrier()
        pltpu.sync_copy(hists_sh, allh_v)

        carry = jnp.zeros((16,), jnp.int32)
        for j in range(15, -1, -1):
            tot = jnp.zeros((16,), jnp.int32)
            wsum = jnp.zeros((16,), jnp.int32)
            for w2 in range(NSUB):
                row = allh_v[w2, pl.ds(j * 16, 16)]
                tot = tot + row
                wsum = jnp.where(w2 < w, wsum + row, wsum)
            r = lax.rev(tot, (0,))
            cs = plsc.cumsum(r)
            excl = cs - r
            gbase = carry + lax.rev(excl, (0,))
            ctr_v[pl.ds(j * 16, 16)] = gbase + wsum
            carry = carry + jnp.max(cs)

        def pbody(ih, _):
            parts = []
            for k in range(2):
                i = ih * 2 + k
                u = su_v[pl.ds(i * 16, 16)]
                digit = ((u >> shift) & jnp.uint32(255)).astype(jnp.int32)
                counts, last = plsc.scan_count(digit)
                parts.append((i, digit, counts, last))
            for i, digit, counts, last in parts:
                ctr = plsc.load_gather(ctr_v, [digit])
                pos = ctr + counts - 1
                pos_v[i // 8, pl.ds((i % 8) * 16, 16)] = pos
                plsc.addupdate_scatter(ctr_v, [digit], counts, mask=last)
            return 0

        lax.fori_loop(0, PERCAP_VREGS // 2, pbody, 0)

        for g in range(0, NWIN, 16):
            copies = []
            for j in range(g, min(g + 16, NWIN)):
                copies.append(pltpu.async_copy(
                    su_v.at[pl.ds(j * 128, 128)], dst_u.at[pos_v.at[j]], sem))
                copies.append(pltpu.async_copy(
                    si_v.at[pl.ds(j * 128, 128)], dst_i.at[pos_v.at[j]], sem))
            for c in copies:
                c.wait()
        plsc.subcore_barrier()

    me = w * NC + cid
    pltpu.sync_copy(bufA_i.at[pl.ds(me * B_PER_W, B_PER_W)],
                    oidx_hbm.at[pl.ds(me * B_PER_W, B_PER_W)])


@functools.partial(
    pl.kernel,
    out_type=(
        jax.ShapeDtypeStruct((OUTPAD,), jnp.float32),
        jax.ShapeDtypeStruct((OUTPAD,), jnp.float32),
    ),
    mesh=_mesh,
    compiler_params=_params,
    scratch_types=[
        pltpu.VMEM((B_PER_W,), jnp.int32),
        pltpu.VMEM((B_PER_W,), jnp.float32),
        pltpu.VMEM((B_PER_W,), jnp.float32),
        pltpu.SemaphoreType.DMA,
        pltpu.SemaphoreType.DMA,
    ],
)
def _gather_kernel(idx_hbm, xs_hbm, ts_hbm, ox_hbm, ot_hbm,
                   idx_v, xrows_v, trows_v, semx, semt):
    w = lax.axis_index("s") * NC + lax.axis_index("c")
    base = w * B_PER_W
    pltpu.sync_copy(idx_hbm.at[pl.ds(base, B_PER_W)], idx_v)
    cx = pltpu.async_copy(xs_hbm.at[idx_v], xrows_v, semx)
    ct = pltpu.async_copy(ts_hbm.at[idx_v], trows_v, semt)
    cx.wait()
    ct.wait()
    pltpu.sync_copy(xrows_v, ox_hbm.at[pl.ds(base, B_PER_W)])
    pltpu.sync_copy(trows_v, ot_hbm.at[pl.ds(base, B_PER_W)])


def kernel(loss, x_s, t_s):
    w = loss.reshape(-1)
    gkey = jax.random.key(42)
    g = jax.random.gumbel(gkey, w.shape, dtype=w.dtype)
    keys = jnp.log(jnp.maximum(w, 1e-30)) + g
    u_mono = _mono_u32_host(keys)
    keys_p = jnp.concatenate(
        [u_mono, jnp.zeros((N_SP - N_S,), jnp.uint32)])

    oidx = _main_kernel(keys_p)
    xg, tg = _gather_kernel(oidx, x_s.reshape(-1), t_s.reshape(-1))
    return (xg[:N].reshape(N, 1), tg[:N].reshape(N, 1))
